# baseline (speedup 1.0000x reference)
import functools

import jax
import jax.numpy as jnp
from jax import lax
from jax.experimental import pallas as pl
from jax.experimental.pallas import tpu as pltpu
from jax.experimental.pallas import tpu_sc as plsc

_ROWS = 16384
_NOUT = 8
_W = 32
_NSC = 4
_NTC = _NOUT - _NSC
_NC = 2
_NS = 16
_NW = _NC * _NS
_CC = _ROWS // _NW
_TR = 512


def _sc_copy_body(xt_hbm, *rest):
    outs = rest[:_NSC]
    bufs = rest[_NSC:_NSC + 2]
    isems = rest[_NSC + 2:_NSC + 4]
    osems = rest[_NSC + 4:]
    wid = lax.axis_index("s") * _NC + lax.axis_index("c")
    c0 = wid * _CC

    def in_cp(i, b):
        return pltpu.make_async_copy(
            xt_hbm.at[pl.ds(i * _W, _W), pl.ds(c0, _CC)], bufs[b], isems[b])

    def out_cp(i, b):
        return pltpu.make_async_copy(
            bufs[b], outs[i].at[:, pl.ds(c0, _CC)], osems[b])

    in_cp(0, 0).start()
    for i in range(_NSC):
        b = i % 2
        if i + 1 < _NSC:
            if i >= 1:
                out_cp(i - 1, 1 - b).wait()
            in_cp(i + 1, 1 - b).start()
        in_cp(i, b).wait()
        out_cp(i, b).start()
    out_cp(_NSC - 2, (_NSC - 2) % 2).wait()
    out_cp(_NSC - 1, (_NSC - 1) % 2).wait()


def _sc_split_t(xt):
    mesh = plsc.VectorSubcoreMesh(core_axis_name="c", subcore_axis_name="s")
    out_type = tuple(
        jax.ShapeDtypeStruct((_W, _ROWS), jnp.float32) for _ in range(_NSC))
    scratch = (
        [pltpu.VMEM((_W, _CC), jnp.float32) for _ in range(2)]
        + [pltpu.SemaphoreType.DMA for _ in range(4)])
    return pl.kernel(
        _sc_copy_body,
        out_type=out_type,
        mesh=mesh,
        scratch_types=scratch,
    )(xt)


def _tc_body(x_hbm, *rest):
    out_refs = rest[:_NTC]
    buf, sem = rest[_NTC], rest[_NTC + 1]
    j = pl.program_id(0)
    cp = pltpu.make_async_copy(
        x_hbm.at[pl.ds(j * _TR, _TR), pl.ds(0, 128)], buf, sem)
    cp.start()
    cp.wait()
    xt = buf[...].T
    for i in range(_NTC):
        out_refs[i][...] = xt[i * _W:(i + 1) * _W, :]


def _tc_split(x):
    grid = (_ROWS // _TR,)
    return pl.pallas_call(
        _tc_body,
        grid=grid,
        in_specs=[pl.BlockSpec(memory_space=pl.ANY)],
        out_specs=[
            pl.BlockSpec((_W, _TR), lambda j: (0, j)) for _ in range(_NTC)],
        out_shape=tuple(
            jax.ShapeDtypeStruct((_W, _ROWS), jnp.float32)
            for _ in range(_NTC)),
        scratch_shapes=[
            pltpu.VMEM((_TR, 128), jnp.float32),
            pltpu.SemaphoreType.DMA,
        ],
    )(x)


@jax.jit
def kernel(x):
    tc_outs = _tc_split(x)
    sc_outs = _sc_split_t(x[:, 128:].T)
    return tuple(o.T for o in tc_outs) + tuple(o.T for o in sc_outs)


# speedup vs baseline: 1.3118x; 1.3118x over previous
import functools

import jax
import jax.numpy as jnp
from jax import lax
from jax.experimental import pallas as pl
from jax.experimental.pallas import tpu as pltpu
from jax.experimental.pallas import tpu_sc as plsc

_ROWS = 16384
_NOUT = 8
_W = 32
_NSC = 4
_NTC = _NOUT - _NSC
_NC = 2
_NS = 16
_NW = _NC * _NS
_CC = _ROWS // _NW
_TR = 512


def _sc_copy_body(xt_hbm, *rest):
    outs = rest[:_NSC]
    bufs = rest[_NSC:_NSC + 2]
    isems = rest[_NSC + 2:_NSC + 4]
    osems = rest[_NSC + 4:]
    wid = lax.axis_index("s") * _NC + lax.axis_index("c")
    c0 = wid * _CC

    def in_cp(i, b):
        return pltpu.make_async_copy(
            xt_hbm.at[pl.ds(i * _W, _W), pl.ds(c0, _CC)], bufs[b], isems[b])

    def out_cp(i, b):
        return pltpu.make_async_copy(
            bufs[b], outs[i].at[:, pl.ds(c0, _CC)], osems[b])

    in_cp(0, 0).start()
    for i in range(_NSC):
        b = i % 2
        if i + 1 < _NSC:
            if i >= 1:
                out_cp(i - 1, 1 - b).wait()
            in_cp(i + 1, 1 - b).start()
        in_cp(i, b).wait()
        out_cp(i, b).start()
    out_cp(_NSC - 2, (_NSC - 2) % 2).wait()
    out_cp(_NSC - 1, (_NSC - 1) % 2).wait()


def _sc_split_t(xt):
    mesh = plsc.VectorSubcoreMesh(core_axis_name="c", subcore_axis_name="s")
    out_type = tuple(
        jax.ShapeDtypeStruct((_W, _ROWS), jnp.float32) for _ in range(_NSC))
    scratch = (
        [pltpu.VMEM((_W, _CC), jnp.float32) for _ in range(2)]
        + [pltpu.SemaphoreType.DMA for _ in range(4)])
    return pl.kernel(
        _sc_copy_body,
        out_type=out_type,
        mesh=mesh,
        scratch_types=scratch,
    )(xt)


def _tc_body(x_ref, *out_refs):
    xt = x_ref[...].T
    for i in range(_NTC):
        out_refs[i][...] = xt[i * _W:(i + 1) * _W, :]


def _tc_split(x):
    x = pltpu.with_memory_space_constraint(x, pltpu.MemorySpace.HBM)
    grid = (_ROWS // _TR,)
    return pl.pallas_call(
        _tc_body,
        grid=grid,
        in_specs=[pl.BlockSpec((_TR, 128), lambda j: (j, 0))],
        out_specs=[
            pl.BlockSpec((_W, _TR), lambda j: (0, j)) for _ in range(_NTC)],
        out_shape=tuple(
            jax.ShapeDtypeStruct((_W, _ROWS), jnp.float32)
            for _ in range(_NTC)),
    )(x)


@jax.jit
def kernel(x):
    tc_outs = _tc_split(x)
    sc_outs = _sc_split_t(x[:, 128:].T)
    return tuple(o.T for o in tc_outs) + tuple(o.T for o in sc_outs)


# speedup vs baseline: 1.5620x; 1.1908x over previous
import functools

import jax
import jax.numpy as jnp
from jax import lax
from jax.experimental import pallas as pl
from jax.experimental.pallas import tpu as pltpu
from jax.experimental.pallas import tpu_sc as plsc

_ROWS = 16384
_NOUT = 8
_W = 32
_NSC = 4
_NTC = _NOUT - _NSC
_NC = 2
_NS = 16
_NW = _NC * _NS
_CC = _ROWS // _NW
_TR = 2048


def _sc_copy_body(xt_hbm, *rest):
    outs = rest[:_NSC]
    bufs = rest[_NSC:_NSC + 2]
    isems = rest[_NSC + 2:_NSC + 4]
    osems = rest[_NSC + 4:]
    wid = lax.axis_index("s") * _NC + lax.axis_index("c")
    c0 = wid * _CC

    def in_cp(i, b):
        return pltpu.make_async_copy(
            xt_hbm.at[pl.ds((_NTC + i) * _W, _W), pl.ds(c0, _CC)],
            bufs[b], isems[b])

    def out_cp(i, b):
        return pltpu.make_async_copy(
            bufs[b], outs[i].at[:, pl.ds(c0, _CC)], osems[b])

    in_cp(0, 0).start()
    for i in range(_NSC):
        b = i % 2
        if i + 1 < _NSC:
            if i >= 1:
                out_cp(i - 1, 1 - b).wait()
            in_cp(i + 1, 1 - b).start()
        in_cp(i, b).wait()
        out_cp(i, b).start()
    out_cp(_NSC - 2, (_NSC - 2) % 2).wait()
    out_cp(_NSC - 1, (_NSC - 1) % 2).wait()


def _sc_split_t(xt):
    mesh = plsc.VectorSubcoreMesh(core_axis_name="c", subcore_axis_name="s")
    out_type = tuple(
        jax.ShapeDtypeStruct((_W, _ROWS), jnp.float32) for _ in range(_NSC))
    scratch = (
        [pltpu.VMEM((_W, _CC), jnp.float32) for _ in range(2)]
        + [pltpu.SemaphoreType.DMA for _ in range(4)])
    return pl.kernel(
        _sc_copy_body,
        out_type=out_type,
        mesh=mesh,
        scratch_types=scratch,
    )(xt)


def _tc_body(x_ref, *out_refs):
    xt = x_ref[...].T
    for i in range(_NTC):
        out_refs[i][...] = xt[i * _W:(i + 1) * _W, :]


def _tc_split(x):
    x = pltpu.with_memory_space_constraint(x, pltpu.MemorySpace.HBM)
    grid = (_ROWS // _TR,)
    return pl.pallas_call(
        _tc_body,
        grid=grid,
        in_specs=[pl.BlockSpec((_TR, 128), lambda j: (j, 0))],
        out_specs=[
            pl.BlockSpec((_W, _TR), lambda j: (0, j)) for _ in range(_NTC)],
        out_shape=tuple(
            jax.ShapeDtypeStruct((_W, _ROWS), jnp.float32)
            for _ in range(_NTC)),
    )(x)


@jax.jit
def kernel(x):
    tc_outs = _tc_split(x)
    sc_outs = _sc_split_t(x.T)
    return tuple(o.T for o in tc_outs) + tuple(o.T for o in sc_outs)
